# baseline (speedup 1.0000x reference)
import functools

import jax
import jax.numpy as jnp
from jax import lax
from jax.experimental import pallas as pl
from jax.experimental.pallas import tpu as pltpu
from jax.experimental.pallas import tpu_sc as plsc

N = 10000
E = 320000
D = 128
H = 256
LD = 64
NUM_SEGMENTS = 256.0

NC = 2
NS = 16
LANES = 16

LOG2PI = 1.8378770664093453


def _sc_mesh():
    return plsc.VectorSubcoreMesh(
        core_axis_name="c", subcore_axis_name="s", num_cores=NC, num_subcores=NS
    )


_EPC = E // NC
_BKT = N // NS
_ACCR = 640
_W = 2000
_NW = _EPC // _W
_CH_A = 80
_CMP = _W + 80 + 112


def _agg_body(src_hbm, dst_hbm, x_hbm, out_hbm, srcw, dstw, csrc, cdst, rows, acc, sem):
    c = lax.axis_index("c")
    s = lax.axis_index("s")
    lo = s * _BKT
    zero16 = jnp.zeros((LANES,), jnp.float32)

    def zrow(r, carry):
        for k in range(D // LANES):
            acc[r, pl.ds(k * LANES, LANES)] = zero16
        return carry

    lax.fori_loop(0, _ACCR, zrow, 0)

    def do_chunk(off):
        pltpu.async_copy(x_hbm.at[csrc.at[pl.ds(off, _CH_A)]], rows, sem).wait()

        def grp(j16, carry):
            dlv = cdst[pl.ds(off + j16 * LANES, LANES)]
            for j in range(LANES):
                dl = dlv[j]
                r = j16 * LANES + j
                for k in range(D // LANES):
                    plsc.addupdate(
                        acc.at[dl, pl.ds(k * LANES, LANES)],
                        rows[r, pl.ds(k * LANES, LANES)],
                    )
            return carry

        lax.fori_loop(0, _CH_A // LANES, grp, 0)

    cbase = c * _EPC

    def win(w, cnt):
        wb = cbase + w * _W
        pltpu.sync_copy(dst_hbm.at[pl.ds(wb, _W)], dstw)
        pltpu.sync_copy(src_hbm.at[pl.ds(wb, _W)], srcw)

        def scan_g(g, cnt2):
            dv = dstw[pl.ds(g * LANES, LANES)]
            sv = srcw[pl.ds(g * LANES, LANES)]
            m = (dv >= lo) & (dv < lo + _BKT)
            plsc.store_compressed(csrc.at[pl.ds(cnt2, LANES)], sv, mask=m)
            plsc.store_compressed(cdst.at[pl.ds(cnt2, LANES)], dv - lo, mask=m)
            pc = plsc.all_reduce_population_count(m)
            return cnt2 + pc[0]

        cnt = lax.fori_loop(0, _W // LANES, scan_g, cnt)
        nfull = cnt // _CH_A

        def chunk_k(k, carry):
            do_chunk(k * _CH_A)
            return carry

        lax.fori_loop(0, nfull, chunk_k, 0)
        left = cnt - nfull * _CH_A
        for j in range(6):
            vs = csrc[pl.ds(nfull * _CH_A + j * LANES, LANES)]
            vd = cdst[pl.ds(nfull * _CH_A + j * LANES, LANES)]
            csrc[pl.ds(j * LANES, LANES)] = vs
            cdst[pl.ds(j * LANES, LANES)] = vd
        return left

    left = lax.fori_loop(0, _NW, win, jnp.int32(0))

    pad_src = lax.iota(jnp.int32, LANES)
    pad_dst = jnp.full((LANES,), _BKT, jnp.int32)
    for j in range(6):
        cdst[pl.ds(left + j * LANES, LANES)] = pad_dst
        csrc[pl.ds(left + j * LANES, LANES)] = pad_src

    @pl.when(left > 0)
    def _():
        do_chunk(0)

    pltpu.sync_copy(acc, out_hbm.at[c, pl.ds(s * _ACCR, _ACCR)])


@functools.lru_cache(maxsize=None)
def _agg_call():
    return functools.partial(
        pl.kernel,
        out_type=jax.ShapeDtypeStruct((NC, NS * _ACCR, D), jnp.float32),
        mesh=_sc_mesh(),
        compiler_params=pltpu.CompilerParams(needs_layout_passes=False),
        scratch_types=[
            pltpu.VMEM((_W,), jnp.int32),
            pltpu.VMEM((_W,), jnp.int32),
            pltpu.VMEM((_CMP,), jnp.int32),
            pltpu.VMEM((_CMP,), jnp.int32),
            pltpu.VMEM((_CH_A, D), jnp.float32),
            pltpu.VMEM((_ACCR, D), jnp.float32),
            pltpu.SemaphoreType.DMA,
        ],
    )(_agg_body)


_RB = 2000
_NB = N // _RB


def _dense_body(p0, p1, x, eps, w1, w2, wmu, wlv, wd, z_out, kl_out, nlp_out):
    i = pl.program_id(0)
    agg = p0[...] + p1[...]
    h = jnp.maximum(
        jnp.dot(agg, w1[...], preferred_element_type=jnp.float32)
        + jnp.dot(x[...], w2[...], preferred_element_type=jnp.float32),
        0.0,
    )
    mu = jnp.dot(h, wmu[...], preferred_element_type=jnp.float32)
    lv = jnp.clip(jnp.dot(h, wlv[...], preferred_element_type=jnp.float32), -8.0, 8.0)
    s2 = jnp.exp(lv)
    z = mu + jnp.exp(0.5 * lv) * eps[...]
    z_out[...] = z
    klb = 0.5 * jnp.sum(mu * mu + s2 - 1.0 - lv)
    xr = jnp.dot(z, wd[...], preferred_element_type=jnp.float32)
    nlb = -0.5 * jnp.sum((x[...] - xr) ** 2) - 0.5 * _RB * D * LOG2PI

    @pl.when(i == 0)
    def _():
        kl_out[0, 0] = klb
        nlp_out[0, 0] = nlb

    @pl.when(i != 0)
    def _():
        kl_out[0, 0] += klb
        nlp_out[0, 0] += nlb


def _dense_call(p0, p1, x, eps, w1, w2, wmu, wlv, wd):
    full = lambda shape: pl.BlockSpec(shape, lambda i: (0, 0))
    blk = lambda shape: pl.BlockSpec(shape, lambda i: (i, 0))
    scalar = pl.BlockSpec((1, 1), lambda i: (0, 0), memory_space=pltpu.SMEM)
    return pl.pallas_call(
        _dense_body,
        grid=(_NB,),
        in_specs=[
            blk((_RB, D)), blk((_RB, D)), blk((_RB, D)), blk((_RB, LD)),
            full((D, H)), full((D, H)), full((H, LD)), full((H, LD)), full((LD, D)),
        ],
        out_specs=[blk((_RB, LD)), scalar, scalar],
        out_shape=[
            jax.ShapeDtypeStruct((N, LD), jnp.float32),
            jax.ShapeDtypeStruct((1, 1), jnp.float32),
            jax.ShapeDtypeStruct((1, 1), jnp.float32),
        ],
    )(p0, p1, x, eps, w1, w2, wmu, wlv, wd)


_CH_C = 80
_EPT = _EPC // NS
_NCH_C = _EPT // _CH_C


def _edge_body(src_hbm, dst_hbm, z_hbm, logit_hbm, sidx, didx, zs, zd, lbuf, sem):
    c = lax.axis_index("c")
    s = lax.axis_index("s")
    base0 = c * _EPC + s * _EPT

    def chunk(i, carry):
        base = base0 + i * _CH_C
        pltpu.sync_copy(src_hbm.at[pl.ds(base, _CH_C)], sidx)
        pltpu.sync_copy(dst_hbm.at[pl.ds(base, _CH_C)], didx)
        pltpu.async_copy(z_hbm.at[sidx], zs, sem).wait()
        pltpu.async_copy(z_hbm.at[didx], zd, sem).wait()

        def egroup(g, carry2):
            rowi = g * LANES + lax.iota(jnp.int32, LANES)

            def dcol(d, acc):
                coli = jnp.full((LANES,), d, jnp.int32)
                a = plsc.load_gather(zs, [rowi, coli])
                b = plsc.load_gather(zd, [rowi, coli])
                return acc + a * b

            acc = lax.fori_loop(0, LD, dcol, jnp.zeros((LANES,), jnp.float32))
            lbuf[pl.ds(g * LANES, LANES)] = acc
            return carry2

        lax.fori_loop(0, _CH_C // LANES, egroup, 0)
        pltpu.sync_copy(lbuf, logit_hbm.at[pl.ds(base, _CH_C)])
        return carry

    lax.fori_loop(0, _NCH_C, chunk, 0)


@functools.lru_cache(maxsize=None)
def _edge_call():
    return functools.partial(
        pl.kernel,
        out_type=jax.ShapeDtypeStruct((E,), jnp.float32),
        mesh=_sc_mesh(),
        compiler_params=pltpu.CompilerParams(
            needs_layout_passes=False, use_tc_tiling_on_sc=False
        ),
        scratch_types=[
            pltpu.VMEM((_CH_C,), jnp.int32),
            pltpu.VMEM((_CH_C,), jnp.int32),
            pltpu.VMEM((_CH_C, LD), jnp.float32),
            pltpu.VMEM((_CH_C, LD), jnp.float32),
            pltpu.VMEM((_CH_C,), jnp.float32),
            pltpu.SemaphoreType.DMA,
        ],
    )(_edge_body)


def _tail_body(l_ref, kl_ref, nlp_ref, out_ref):
    t = l_ref[...]
    elp = jnp.sum(jnp.minimum(t, 0.0) - jnp.log1p(jnp.exp(-jnp.abs(t))))
    out_ref[0, 0] = -((nlp_ref[0, 0] + elp - kl_ref[0, 0]) / NUM_SEGMENTS)


def _tail_call(logits2d, kl_s, nlp_s):
    scalar = pl.BlockSpec(memory_space=pltpu.SMEM)
    return pl.pallas_call(
        _tail_body,
        in_specs=[pl.BlockSpec(logits2d.shape, lambda: (0, 0)), scalar, scalar],
        out_specs=scalar,
        out_shape=jax.ShapeDtypeStruct((1, 1), jnp.float32),
    )(logits2d, kl_s, nlp_s)


def kernel(x, edge_index, batch, eps, W1, W2, Wmu, Wlv, Wd):
    del batch
    src = edge_index[0]
    dst = edge_index[1]
    parts = _agg_call()(src, dst, x)
    p = parts.reshape(NC, NS, _ACCR, D)[:, :, :_BKT, :].reshape(NC, N, D)
    z, kl_s, nlp_s = _dense_call(p[0], p[1], x, eps, W1, W2, Wmu, Wlv, Wd)
    logits = _edge_call()(src, dst, z)
    out = _tail_call(logits.reshape(E // D, D), kl_s, nlp_s)
    return out[0, 0]


# speedup vs baseline: 1.9183x; 1.9183x over previous
import functools

import jax
import jax.numpy as jnp
from jax import lax
from jax.experimental import pallas as pl
from jax.experimental.pallas import tpu as pltpu
from jax.experimental.pallas import tpu_sc as plsc

N = 10000
E = 320000
D = 128
H = 256
LD = 64
NUM_SEGMENTS = 256.0

NC = 2
NS = 16
LANES = 16

LOG2PI = 1.8378770664093453


def _sc_mesh():
    return plsc.VectorSubcoreMesh(
        core_axis_name="c", subcore_axis_name="s", num_cores=NC, num_subcores=NS
    )


_EPC = E // NC
_BLK = 8
_RSTRIDE = 624
_RWIN = 640


def _tile_range(c, s, bpc):
    nhi = bpc % NS
    nlo = bpc // NS
    nb = jnp.where(s < nhi, nlo + 1, nlo)
    start = c * bpc + jnp.where(s < nhi, s * (nlo + 1), nhi * (nlo + 1) + (s - nhi) * nlo)
    return start, nb


def _load_idx(src2_hbm, dst2_hbm, swin, dwin, b0, nb, nlo):
    r0 = b0 * _BLK
    pltpu.sync_copy(src2_hbm.at[pl.ds(r0, nlo * _BLK)], swin.at[pl.ds(0, nlo * _BLK)])
    pltpu.sync_copy(dst2_hbm.at[pl.ds(r0, nlo * _BLK)], dwin.at[pl.ds(0, nlo * _BLK)])

    @pl.when(nb > nlo)
    def _():
        pltpu.sync_copy(src2_hbm.at[pl.ds(r0 + nlo * _BLK, _BLK)],
                        swin.at[pl.ds(nlo * _BLK, _BLK)])
        pltpu.sync_copy(dst2_hbm.at[pl.ds(r0 + nlo * _BLK, _BLK)],
                        dwin.at[pl.ds(nlo * _BLK, _BLK)])


_CHA = 40
_ERA = E // _CHA
_BPCA = _ERA // NC // _BLK
_NLOA = _BPCA // NS
_MAXCHA = (_NLOA + 1) * _BLK


def _agg_body(src2_hbm, dst2_hbm, x_hbm, zeros_hbm, out_hbm,
              swin, dwin, rows0, rows1, acc, g0, g1, s0, s1):
    c = lax.axis_index("c")
    s = lax.axis_index("s")
    pltpu.sync_copy(
        zeros_hbm.at[pl.ds(s * _RSTRIDE, _RWIN)], acc.at[pl.ds(s * _RSTRIDE, _RWIN)]
    )
    b0, nb = _tile_range(c, s, _BPCA)
    nchunk = nb * _BLK
    _load_idx(src2_hbm, dst2_hbm, swin, dwin, b0, nb, _NLOA)
    plsc.subcore_barrier()

    pltpu.async_copy(x_hbm.at[swin.at[0]], rows0, g0)

    def chunk(k, carry):
        @pl.when(k % 2 == 0)
        def _():
            @pl.when(k >= 1)
            def _():
                pltpu.make_async_copy(rows1, acc.at[dwin.at[k - 1]], s1).wait()

            @pl.when(k + 1 < nchunk)
            def _():
                pltpu.async_copy(x_hbm.at[swin.at[k + 1]], rows1, g1)

            pltpu.make_async_copy(x_hbm.at[swin.at[k]], rows0, g0).wait()
            pltpu.async_copy(rows0, acc.at[dwin.at[k]], s0, add=True)

        @pl.when(k % 2 == 1)
        def _():
            pltpu.make_async_copy(rows0, acc.at[dwin.at[k - 1]], s0).wait()

            @pl.when(k + 1 < nchunk)
            def _():
                pltpu.async_copy(x_hbm.at[swin.at[k + 1]], rows0, g0)

            pltpu.make_async_copy(x_hbm.at[swin.at[k]], rows1, g1).wait()
            pltpu.async_copy(rows1, acc.at[dwin.at[k]], s1, add=True)

        return carry

    lax.fori_loop(0, nchunk, chunk, 0)
    pltpu.make_async_copy(rows1, acc.at[dwin.at[nchunk - 1]], s1).wait()
    plsc.subcore_barrier()
    pltpu.sync_copy(
        acc.at[pl.ds(s * _RSTRIDE, _RWIN)], out_hbm.at[c, pl.ds(s * _RSTRIDE, _RWIN)]
    )


@functools.lru_cache(maxsize=None)
def _agg_call():
    return functools.partial(
        pl.kernel,
        out_type=jax.ShapeDtypeStruct((NC, N, D), jnp.float32),
        mesh=_sc_mesh(),
        compiler_params=pltpu.CompilerParams(
            needs_layout_passes=False, use_tc_tiling_on_sc=False
        ),
        scratch_types=[
            pltpu.VMEM((_MAXCHA, _CHA), jnp.int32),
            pltpu.VMEM((_MAXCHA, _CHA), jnp.int32),
            pltpu.VMEM((_CHA, D), jnp.float32),
            pltpu.VMEM((_CHA, D), jnp.float32),
            pltpu.VMEM_SHARED((N, D), jnp.float32),
            pltpu.SemaphoreType.DMA,
            pltpu.SemaphoreType.DMA,
            pltpu.SemaphoreType.DMA,
            pltpu.SemaphoreType.DMA,
        ],
    )(_agg_body)


_RB = 2000
_NB = N // _RB


def _dense_body(p0, p1, x, eps, w1, w2, wmu, wlv, wd, z_out, kl_out, nlp_out):
    i = pl.program_id(0)
    agg = p0[...] + p1[...]
    h = jnp.maximum(
        jnp.dot(agg, w1[...], preferred_element_type=jnp.float32)
        + jnp.dot(x[...], w2[...], preferred_element_type=jnp.float32),
        0.0,
    )
    mu = jnp.dot(h, wmu[...], preferred_element_type=jnp.float32)
    lv = jnp.clip(jnp.dot(h, wlv[...], preferred_element_type=jnp.float32), -8.0, 8.0)
    s2 = jnp.exp(lv)
    z = mu + jnp.exp(0.5 * lv) * eps[...]
    z_out[...] = z
    klb = 0.5 * jnp.sum(mu * mu + s2 - 1.0 - lv)
    xr = jnp.dot(z, wd[...], preferred_element_type=jnp.float32)
    nlb = -0.5 * jnp.sum((x[...] - xr) ** 2) - 0.5 * _RB * D * LOG2PI

    @pl.when(i == 0)
    def _():
        kl_out[0, 0] = klb
        nlp_out[0, 0] = nlb

    @pl.when(i != 0)
    def _():
        kl_out[0, 0] += klb
        nlp_out[0, 0] += nlb


def _dense_call(p0, p1, x, eps, w1, w2, wmu, wlv, wd):
    full = lambda shape: pl.BlockSpec(shape, lambda i: (0, 0))
    blk = lambda shape: pl.BlockSpec(shape, lambda i: (i, 0))
    scalar = pl.BlockSpec((1, 1), lambda i: (0, 0), memory_space=pltpu.SMEM)
    return pl.pallas_call(
        _dense_body,
        grid=(_NB,),
        in_specs=[
            blk((_RB, D)), blk((_RB, D)), blk((_RB, D)), blk((_RB, LD)),
            full((D, H)), full((D, H)), full((H, LD)), full((H, LD)), full((LD, D)),
        ],
        out_specs=[blk((_RB, LD)), scalar, scalar],
        out_shape=[
            jax.ShapeDtypeStruct((N, LD), jnp.float32),
            jax.ShapeDtypeStruct((1, 1), jnp.float32),
            jax.ShapeDtypeStruct((1, 1), jnp.float32),
        ],
    )(p0, p1, x, eps, w1, w2, wmu, wlv, wd)


_CHC = 80
_ERC = E // _CHC
_BPCC = _ERC // NC // _BLK
_NLOC = _BPCC // NS
_MAXCHC = (_NLOC + 1) * _BLK


def _edge_dot_chunk(zs, zd, lbuf, k):
    for g in range(_CHC // LANES):
        rowi = g * LANES + lax.iota(jnp.int32, LANES)

        def dblk(db, accs):
            a0, a1 = accs
            for dd in range(0, LANES, 2):
                c0 = jnp.full((LANES,), db * LANES + dd, jnp.int32)
                c1 = c0 + 1
                a0 = a0 + plsc.load_gather(zs, [rowi, c0]) * plsc.load_gather(zd, [rowi, c0])
                a1 = a1 + plsc.load_gather(zs, [rowi, c1]) * plsc.load_gather(zd, [rowi, c1])
            return a0, a1

        z16 = jnp.zeros((LANES,), jnp.float32)
        a0, a1 = lax.fori_loop(0, LD // LANES, dblk, (z16, z16))
        lbuf[pl.ds(k * _CHC + g * LANES, LANES)] = a0 + a1


def _edge_body(src2_hbm, dst2_hbm, z_hbm, logit_hbm,
               swin, dwin, zs0, zd0, zs1, zd1, lbuf, g0, g1):
    c = lax.axis_index("c")
    s = lax.axis_index("s")
    b0, nb = _tile_range(c, s, _BPCC)
    nchunk = nb * _BLK
    _load_idx(src2_hbm, dst2_hbm, swin, dwin, b0, nb, _NLOC)

    pltpu.async_copy(z_hbm.at[swin.at[0]], zs0, g0)
    pltpu.async_copy(z_hbm.at[dwin.at[0]], zd0, g0)

    def chunk(k, carry):
        @pl.when(k % 2 == 0)
        def _():
            @pl.when(k + 1 < nchunk)
            def _():
                pltpu.async_copy(z_hbm.at[swin.at[k + 1]], zs1, g1)
                pltpu.async_copy(z_hbm.at[dwin.at[k + 1]], zd1, g1)

            pltpu.make_async_copy(z_hbm.at[swin.at[k]], zs0, g0).wait()
            pltpu.make_async_copy(z_hbm.at[dwin.at[k]], zd0, g0).wait()
            _edge_dot_chunk(zs0, zd0, lbuf, k)

        @pl.when(k % 2 == 1)
        def _():
            @pl.when(k + 1 < nchunk)
            def _():
                pltpu.async_copy(z_hbm.at[swin.at[k + 1]], zs0, g0)
                pltpu.async_copy(z_hbm.at[dwin.at[k + 1]], zd0, g0)

            pltpu.make_async_copy(z_hbm.at[swin.at[k]], zs1, g1).wait()
            pltpu.make_async_copy(z_hbm.at[dwin.at[k]], zd1, g1).wait()
            _edge_dot_chunk(zs1, zd1, lbuf, k)

        return carry

    lax.fori_loop(0, nchunk, chunk, 0)

    e0 = b0 * _BLK * _CHC
    nfix = _NLOC * _BLK * _CHC
    pltpu.sync_copy(lbuf.at[pl.ds(0, nfix)], logit_hbm.at[pl.ds(e0, nfix)])

    @pl.when(nchunk * _CHC > nfix)
    def _():
        pltpu.sync_copy(
            lbuf.at[pl.ds(nfix, _BLK * _CHC)],
            logit_hbm.at[pl.ds(e0 + nfix, _BLK * _CHC)],
        )


@functools.lru_cache(maxsize=None)
def _edge_call():
    return functools.partial(
        pl.kernel,
        out_type=jax.ShapeDtypeStruct((E,), jnp.float32),
        mesh=_sc_mesh(),
        compiler_params=pltpu.CompilerParams(
            needs_layout_passes=False, use_tc_tiling_on_sc=False
        ),
        scratch_types=[
            pltpu.VMEM((_MAXCHC, _CHC), jnp.int32),
            pltpu.VMEM((_MAXCHC, _CHC), jnp.int32),
            pltpu.VMEM((_CHC, LD), jnp.float32),
            pltpu.VMEM((_CHC, LD), jnp.float32),
            pltpu.VMEM((_CHC, LD), jnp.float32),
            pltpu.VMEM((_CHC, LD), jnp.float32),
            pltpu.VMEM((_MAXCHC * _CHC,), jnp.float32),
            pltpu.SemaphoreType.DMA,
            pltpu.SemaphoreType.DMA,
        ],
    )(_edge_body)


def _tail_body(l_ref, kl_ref, nlp_ref, out_ref):
    t = l_ref[...]
    elp = jnp.sum(jnp.minimum(t, 0.0) - jnp.log1p(jnp.exp(-jnp.abs(t))))
    out_ref[0, 0] = -((nlp_ref[0, 0] + elp - kl_ref[0, 0]) / NUM_SEGMENTS)


def _tail_call(logits2d, kl_s, nlp_s):
    scalar = pl.BlockSpec(memory_space=pltpu.SMEM)
    return pl.pallas_call(
        _tail_body,
        in_specs=[pl.BlockSpec(logits2d.shape, lambda: (0, 0)), scalar, scalar],
        out_specs=scalar,
        out_shape=jax.ShapeDtypeStruct((1, 1), jnp.float32),
    )(logits2d, kl_s, nlp_s)


def kernel(x, edge_index, batch, eps, W1, W2, Wmu, Wlv, Wd):
    del batch
    src = edge_index[0]
    dst = edge_index[1]
    zeros = jnp.zeros((N, D), jnp.float32)
    parts = _agg_call()(src.reshape(_ERA, _CHA), dst.reshape(_ERA, _CHA), x, zeros)
    z, kl_s, nlp_s = _dense_call(parts[0], parts[1], x, eps, W1, W2, Wmu, Wlv, Wd)
    logits = _edge_call()(src.reshape(_ERC, _CHC), dst.reshape(_ERC, _CHC), z)
    out = _tail_call(logits.reshape(E // D, D), kl_s, nlp_s)
    return out[0, 0]


# speedup vs baseline: 2.0025x; 1.0439x over previous
import functools

import jax
import jax.numpy as jnp
from jax import lax
from jax.experimental import pallas as pl
from jax.experimental.pallas import tpu as pltpu
from jax.experimental.pallas import tpu_sc as plsc

N = 10000
E = 320000
D = 128
H = 256
LD = 64
NUM_SEGMENTS = 256.0

NC = 2
NS = 16
LANES = 16

LOG2PI = 1.8378770664093453


def _sc_mesh():
    return plsc.VectorSubcoreMesh(
        core_axis_name="c", subcore_axis_name="s", num_cores=NC, num_subcores=NS
    )


_EPC = E // NC
_BLK = 8
_RSTRIDE = 624
_RWIN = 640


def _tile_range(c, s, bpc):
    nhi = bpc % NS
    nlo = bpc // NS
    nb = jnp.where(s < nhi, nlo + 1, nlo)
    start = c * bpc + jnp.where(s < nhi, s * (nlo + 1), nhi * (nlo + 1) + (s - nhi) * nlo)
    return start, nb


def _load_idx(src2_hbm, dst2_hbm, swin, dwin, b0, nb, nlo):
    r0 = b0 * _BLK
    pltpu.sync_copy(src2_hbm.at[pl.ds(r0, nlo * _BLK)], swin.at[pl.ds(0, nlo * _BLK)])
    pltpu.sync_copy(dst2_hbm.at[pl.ds(r0, nlo * _BLK)], dwin.at[pl.ds(0, nlo * _BLK)])

    @pl.when(nb > nlo)
    def _():
        pltpu.sync_copy(src2_hbm.at[pl.ds(r0 + nlo * _BLK, _BLK)],
                        swin.at[pl.ds(nlo * _BLK, _BLK)])
        pltpu.sync_copy(dst2_hbm.at[pl.ds(r0 + nlo * _BLK, _BLK)],
                        dwin.at[pl.ds(nlo * _BLK, _BLK)])


_CHA = 40
_ERA = E // _CHA
_BPCA = _ERA // NC // _BLK
_NLOA = _BPCA // NS
_MAXCHA = (_NLOA + 1) * _BLK


def _agg_body(src2_hbm, dst2_hbm, x_hbm, zeros_hbm, out_hbm,
              swin, dwin, rows0, rows1, acc, g0, g1, s0, s1):
    c = lax.axis_index("c")
    s = lax.axis_index("s")
    pltpu.sync_copy(
        zeros_hbm.at[pl.ds(s * _RSTRIDE, _RWIN)], acc.at[pl.ds(s * _RSTRIDE, _RWIN)]
    )
    b0, nb = _tile_range(c, s, _BPCA)
    nchunk = nb * _BLK
    _load_idx(src2_hbm, dst2_hbm, swin, dwin, b0, nb, _NLOA)
    plsc.subcore_barrier()

    pltpu.async_copy(x_hbm.at[swin.at[0]], rows0, g0)

    def chunk(k, carry):
        @pl.when(k % 2 == 0)
        def _():
            @pl.when(k >= 1)
            def _():
                pltpu.make_async_copy(rows1, acc.at[dwin.at[k - 1]], s1).wait()

            @pl.when(k + 1 < nchunk)
            def _():
                pltpu.async_copy(x_hbm.at[swin.at[k + 1]], rows1, g1)

            pltpu.make_async_copy(x_hbm.at[swin.at[k]], rows0, g0).wait()
            pltpu.async_copy(rows0, acc.at[dwin.at[k]], s0, add=True)

        @pl.when(k % 2 == 1)
        def _():
            pltpu.make_async_copy(rows0, acc.at[dwin.at[k - 1]], s0).wait()

            @pl.when(k + 1 < nchunk)
            def _():
                pltpu.async_copy(x_hbm.at[swin.at[k + 1]], rows0, g0)

            pltpu.make_async_copy(x_hbm.at[swin.at[k]], rows1, g1).wait()
            pltpu.async_copy(rows1, acc.at[dwin.at[k]], s1, add=True)

        return carry

    lax.fori_loop(0, nchunk, chunk, 0)
    pltpu.make_async_copy(rows1, acc.at[dwin.at[nchunk - 1]], s1).wait()
    plsc.subcore_barrier()
    pltpu.sync_copy(
        acc.at[pl.ds(s * _RSTRIDE, _RWIN)], out_hbm.at[c, pl.ds(s * _RSTRIDE, _RWIN)]
    )


@functools.lru_cache(maxsize=None)
def _agg_call():
    return functools.partial(
        pl.kernel,
        out_type=jax.ShapeDtypeStruct((NC, N, D), jnp.float32),
        mesh=_sc_mesh(),
        compiler_params=pltpu.CompilerParams(
            needs_layout_passes=False, use_tc_tiling_on_sc=False
        ),
        scratch_types=[
            pltpu.VMEM((_MAXCHA, _CHA), jnp.int32),
            pltpu.VMEM((_MAXCHA, _CHA), jnp.int32),
            pltpu.VMEM((_CHA, D), jnp.float32),
            pltpu.VMEM((_CHA, D), jnp.float32),
            pltpu.VMEM_SHARED((N, D), jnp.float32),
            pltpu.SemaphoreType.DMA,
            pltpu.SemaphoreType.DMA,
            pltpu.SemaphoreType.DMA,
            pltpu.SemaphoreType.DMA,
        ],
    )(_agg_body)


_RB = 2000
_NB = N // _RB


def _dense_body(p0, p1, x, eps, w1, w2, wmu, wlv, wd, z_out, kl_out, nlp_out):
    i = pl.program_id(0)
    agg = p0[...] + p1[...]
    h = jnp.maximum(
        jnp.dot(agg, w1[...], preferred_element_type=jnp.float32)
        + jnp.dot(x[...], w2[...], preferred_element_type=jnp.float32),
        0.0,
    )
    mu = jnp.dot(h, wmu[...], preferred_element_type=jnp.float32)
    lv = jnp.clip(jnp.dot(h, wlv[...], preferred_element_type=jnp.float32), -8.0, 8.0)
    s2 = jnp.exp(lv)
    z = mu + jnp.exp(0.5 * lv) * eps[...]
    z_out[...] = z
    klb = 0.5 * jnp.sum(mu * mu + s2 - 1.0 - lv)
    xr = jnp.dot(z, wd[...], preferred_element_type=jnp.float32)
    nlb = -0.5 * jnp.sum((x[...] - xr) ** 2) - 0.5 * _RB * D * LOG2PI

    @pl.when(i == 0)
    def _():
        kl_out[0, 0] = klb
        nlp_out[0, 0] = nlb

    @pl.when(i != 0)
    def _():
        kl_out[0, 0] += klb
        nlp_out[0, 0] += nlb


def _dense_call(p0, p1, x, eps, w1, w2, wmu, wlv, wd):
    full = lambda shape: pl.BlockSpec(shape, lambda i: (0, 0))
    blk = lambda shape: pl.BlockSpec(shape, lambda i: (i, 0))
    scalar = pl.BlockSpec((1, 1), lambda i: (0, 0), memory_space=pltpu.SMEM)
    return pl.pallas_call(
        _dense_body,
        grid=(_NB,),
        in_specs=[
            blk((_RB, D)), blk((_RB, D)), blk((_RB, D)), blk((_RB, LD)),
            full((D, H)), full((D, H)), full((H, LD)), full((H, LD)), full((LD, D)),
        ],
        out_specs=[blk((_RB, LD)), scalar, scalar],
        out_shape=[
            jax.ShapeDtypeStruct((N, LD), jnp.float32),
            jax.ShapeDtypeStruct((1, 1), jnp.float32),
            jax.ShapeDtypeStruct((1, 1), jnp.float32),
        ],
    )(p0, p1, x, eps, w1, w2, wmu, wlv, wd)


_CHC = 80
_ERC = E // _CHC
_BPCC = _ERC // NC // _BLK
_NLOC = _BPCC // NS
_MAXCHC = (_NLOC + 1) * _BLK


def _edge_dot_chunk(zs, zd, lbuf, k):
    m0 = lax.iota(jnp.int32, LANES) == 0
    base = k * _CHC
    for e in range(_CHC):
        p0 = zs[e, pl.ds(0, 16)] * zd[e, pl.ds(0, 16)]
        p1 = zs[e, pl.ds(16, 16)] * zd[e, pl.ds(16, 16)]
        p2 = zs[e, pl.ds(32, 16)] * zd[e, pl.ds(32, 16)]
        p3 = zs[e, pl.ds(48, 16)] * zd[e, pl.ds(48, 16)]
        sv = jnp.sum((p0 + p1) + (p2 + p3))
        plsc.store_scatter(
            lbuf,
            [jnp.full((LANES,), base + e, jnp.int32)],
            jnp.full((LANES,), sv, jnp.float32),
            mask=m0,
        )


def _edge_body(src2_hbm, dst2_hbm, z_hbm, logit_hbm,
               swin, dwin, zs0, zd0, zs1, zd1, lbuf, g0, g1):
    c = lax.axis_index("c")
    s = lax.axis_index("s")
    b0, nb = _tile_range(c, s, _BPCC)
    nchunk = nb * _BLK
    _load_idx(src2_hbm, dst2_hbm, swin, dwin, b0, nb, _NLOC)

    pltpu.async_copy(z_hbm.at[swin.at[0]], zs0, g0)
    pltpu.async_copy(z_hbm.at[dwin.at[0]], zd0, g0)

    def chunk(k, carry):
        @pl.when(k % 2 == 0)
        def _():
            @pl.when(k + 1 < nchunk)
            def _():
                pltpu.async_copy(z_hbm.at[swin.at[k + 1]], zs1, g1)
                pltpu.async_copy(z_hbm.at[dwin.at[k + 1]], zd1, g1)

            pltpu.make_async_copy(z_hbm.at[swin.at[k]], zs0, g0).wait()
            pltpu.make_async_copy(z_hbm.at[dwin.at[k]], zd0, g0).wait()
            _edge_dot_chunk(zs0, zd0, lbuf, k)

        @pl.when(k % 2 == 1)
        def _():
            @pl.when(k + 1 < nchunk)
            def _():
                pltpu.async_copy(z_hbm.at[swin.at[k + 1]], zs0, g0)
                pltpu.async_copy(z_hbm.at[dwin.at[k + 1]], zd0, g0)

            pltpu.make_async_copy(z_hbm.at[swin.at[k]], zs1, g1).wait()
            pltpu.make_async_copy(z_hbm.at[dwin.at[k]], zd1, g1).wait()
            _edge_dot_chunk(zs1, zd1, lbuf, k)

        return carry

    lax.fori_loop(0, nchunk, chunk, 0)

    e0 = b0 * _BLK * _CHC
    nfix = _NLOC * _BLK * _CHC
    pltpu.sync_copy(lbuf.at[pl.ds(0, nfix)], logit_hbm.at[pl.ds(e0, nfix)])

    @pl.when(nchunk * _CHC > nfix)
    def _():
        pltpu.sync_copy(
            lbuf.at[pl.ds(nfix, _BLK * _CHC)],
            logit_hbm.at[pl.ds(e0 + nfix, _BLK * _CHC)],
        )


@functools.lru_cache(maxsize=None)
def _edge_call():
    return functools.partial(
        pl.kernel,
        out_type=jax.ShapeDtypeStruct((E,), jnp.float32),
        mesh=_sc_mesh(),
        compiler_params=pltpu.CompilerParams(
            needs_layout_passes=False, use_tc_tiling_on_sc=False
        ),
        scratch_types=[
            pltpu.VMEM((_MAXCHC, _CHC), jnp.int32),
            pltpu.VMEM((_MAXCHC, _CHC), jnp.int32),
            pltpu.VMEM((_CHC, LD), jnp.float32),
            pltpu.VMEM((_CHC, LD), jnp.float32),
            pltpu.VMEM((_CHC, LD), jnp.float32),
            pltpu.VMEM((_CHC, LD), jnp.float32),
            pltpu.VMEM((_MAXCHC * _CHC,), jnp.float32),
            pltpu.SemaphoreType.DMA,
            pltpu.SemaphoreType.DMA,
        ],
    )(_edge_body)


def _tail_body(l_ref, kl_ref, nlp_ref, out_ref):
    t = l_ref[...]
    elp = jnp.sum(jnp.minimum(t, 0.0) - jnp.log1p(jnp.exp(-jnp.abs(t))))
    out_ref[0, 0] = -((nlp_ref[0, 0] + elp - kl_ref[0, 0]) / NUM_SEGMENTS)


def _tail_call(logits2d, kl_s, nlp_s):
    scalar = pl.BlockSpec(memory_space=pltpu.SMEM)
    return pl.pallas_call(
        _tail_body,
        in_specs=[pl.BlockSpec(logits2d.shape, lambda: (0, 0)), scalar, scalar],
        out_specs=scalar,
        out_shape=jax.ShapeDtypeStruct((1, 1), jnp.float32),
    )(logits2d, kl_s, nlp_s)


def kernel(x, edge_index, batch, eps, W1, W2, Wmu, Wlv, Wd):
    del batch
    src = edge_index[0]
    dst = edge_index[1]
    zeros = jnp.zeros((N, D), jnp.float32)
    parts = _agg_call()(src.reshape(_ERA, _CHA), dst.reshape(_ERA, _CHA), x, zeros)
    z, kl_s, nlp_s = _dense_call(parts[0], parts[1], x, eps, W1, W2, Wmu, Wlv, Wd)
    logits = _edge_call()(src.reshape(_ERC, _CHC), dst.reshape(_ERC, _CHC), z)
    out = _tail_call(logits.reshape(E // D, D), kl_s, nlp_s)
    return out[0, 0]


# speedup vs baseline: 2.2604x; 1.1288x over previous
import functools

import jax
import jax.numpy as jnp
from jax import lax
from jax.experimental import pallas as pl
from jax.experimental.pallas import tpu as pltpu
from jax.experimental.pallas import tpu_sc as plsc

N = 10000
E = 320000
D = 128
H = 256
LD = 64
NUM_SEGMENTS = 256.0

NC = 2
NS = 16
LANES = 16

LOG2PI = 1.8378770664093453


def _sc_mesh():
    return plsc.VectorSubcoreMesh(
        core_axis_name="c", subcore_axis_name="s", num_cores=NC, num_subcores=NS
    )


_EPC = E // NC
_BLK = 8
_RSTRIDE = 624
_RWIN = 640


def _tile_range(c, s, bpc):
    nhi = bpc % NS
    nlo = bpc // NS
    nb = jnp.where(s < nhi, nlo + 1, nlo)
    start = c * bpc + jnp.where(s < nhi, s * (nlo + 1), nhi * (nlo + 1) + (s - nhi) * nlo)
    return start, nb


def _load_idx(src2_hbm, dst2_hbm, swin, dwin, b0, nb, nlo):
    r0 = b0 * _BLK
    pltpu.sync_copy(src2_hbm.at[pl.ds(r0, nlo * _BLK)], swin.at[pl.ds(0, nlo * _BLK)])
    pltpu.sync_copy(dst2_hbm.at[pl.ds(r0, nlo * _BLK)], dwin.at[pl.ds(0, nlo * _BLK)])

    @pl.when(nb > nlo)
    def _():
        pltpu.sync_copy(src2_hbm.at[pl.ds(r0 + nlo * _BLK, _BLK)],
                        swin.at[pl.ds(nlo * _BLK, _BLK)])
        pltpu.sync_copy(dst2_hbm.at[pl.ds(r0 + nlo * _BLK, _BLK)],
                        dwin.at[pl.ds(nlo * _BLK, _BLK)])


_CHA = 40
_ERA = E // _CHA
_BPCA = _ERA // NC // _BLK
_NLOA = _BPCA // NS
_MAXCHA = (_NLOA + 1) * _BLK


def _agg_body(src2_hbm, dst2_hbm, x_hbm, zeros_hbm, out_hbm,
              swin, dwin, rows0, rows1, acc, g0, g1, s0, s1):
    c = lax.axis_index("c")
    s = lax.axis_index("s")
    pltpu.sync_copy(
        zeros_hbm.at[pl.ds(s * _RSTRIDE, _RWIN)], acc.at[pl.ds(s * _RSTRIDE, _RWIN)]
    )
    b0, nb = _tile_range(c, s, _BPCA)
    nchunk = nb * _BLK
    _load_idx(src2_hbm, dst2_hbm, swin, dwin, b0, nb, _NLOA)
    plsc.subcore_barrier()

    pltpu.async_copy(x_hbm.at[swin.at[0]], rows0, g0)

    def chunk(k, carry):
        @pl.when(k % 2 == 0)
        def _():
            @pl.when(k >= 1)
            def _():
                pltpu.make_async_copy(rows1, acc.at[dwin.at[k - 1]], s1).wait()

            @pl.when(k + 1 < nchunk)
            def _():
                pltpu.async_copy(x_hbm.at[swin.at[k + 1]], rows1, g1)

            pltpu.make_async_copy(x_hbm.at[swin.at[k]], rows0, g0).wait()
            pltpu.async_copy(rows0, acc.at[dwin.at[k]], s0, add=True)

        @pl.when(k % 2 == 1)
        def _():
            pltpu.make_async_copy(rows0, acc.at[dwin.at[k - 1]], s0).wait()

            @pl.when(k + 1 < nchunk)
            def _():
                pltpu.async_copy(x_hbm.at[swin.at[k + 1]], rows0, g0)

            pltpu.make_async_copy(x_hbm.at[swin.at[k]], rows1, g1).wait()
            pltpu.async_copy(rows1, acc.at[dwin.at[k]], s1, add=True)

        return carry

    lax.fori_loop(0, nchunk, chunk, 0)
    pltpu.make_async_copy(rows1, acc.at[dwin.at[nchunk - 1]], s1).wait()
    plsc.subcore_barrier()
    pltpu.sync_copy(
        acc.at[pl.ds(s * _RSTRIDE, _RWIN)], out_hbm.at[c, pl.ds(s * _RSTRIDE, _RWIN)]
    )


@functools.lru_cache(maxsize=None)
def _agg_call():
    return functools.partial(
        pl.kernel,
        out_type=jax.ShapeDtypeStruct((NC, N, D), jnp.float32),
        mesh=_sc_mesh(),
        compiler_params=pltpu.CompilerParams(
            needs_layout_passes=False, use_tc_tiling_on_sc=False
        ),
        scratch_types=[
            pltpu.VMEM((_MAXCHA, _CHA), jnp.int32),
            pltpu.VMEM((_MAXCHA, _CHA), jnp.int32),
            pltpu.VMEM((_CHA, D), jnp.float32),
            pltpu.VMEM((_CHA, D), jnp.float32),
            pltpu.VMEM_SHARED((N, D), jnp.float32),
            pltpu.SemaphoreType.DMA,
            pltpu.SemaphoreType.DMA,
            pltpu.SemaphoreType.DMA,
            pltpu.SemaphoreType.DMA,
        ],
    )(_agg_body)


_RB = 2000
_NB = N // _RB


def _dense_body(p0, p1, x, eps, w1, w2, wmu, wlv, wd, z_out, kl_out, nlp_out):
    i = pl.program_id(0)
    agg = p0[...] + p1[...]
    h = jnp.maximum(
        jnp.dot(agg, w1[...], preferred_element_type=jnp.float32)
        + jnp.dot(x[...], w2[...], preferred_element_type=jnp.float32),
        0.0,
    )
    mu = jnp.dot(h, wmu[...], preferred_element_type=jnp.float32)
    lv = jnp.clip(jnp.dot(h, wlv[...], preferred_element_type=jnp.float32), -8.0, 8.0)
    s2 = jnp.exp(lv)
    z = mu + jnp.exp(0.5 * lv) * eps[...]
    z_out[...] = z
    klb = 0.5 * jnp.sum(mu * mu + s2 - 1.0 - lv)
    xr = jnp.dot(z, wd[...], preferred_element_type=jnp.float32)
    nlb = -0.5 * jnp.sum((x[...] - xr) ** 2) - 0.5 * _RB * D * LOG2PI

    @pl.when(i == 0)
    def _():
        kl_out[0, 0] = klb
        nlp_out[0, 0] = nlb

    @pl.when(i != 0)
    def _():
        kl_out[0, 0] += klb
        nlp_out[0, 0] += nlb


def _dense_call(p0, p1, x, eps, w1, w2, wmu, wlv, wd):
    full = lambda shape: pl.BlockSpec(shape, lambda i: (0, 0))
    blk = lambda shape: pl.BlockSpec(shape, lambda i: (i, 0))
    scalar = pl.BlockSpec((1, 1), lambda i: (0, 0), memory_space=pltpu.SMEM)
    return pl.pallas_call(
        _dense_body,
        grid=(_NB,),
        in_specs=[
            blk((_RB, D)), blk((_RB, D)), blk((_RB, D)), blk((_RB, LD)),
            full((D, H)), full((D, H)), full((H, LD)), full((H, LD)), full((LD, D)),
        ],
        out_specs=[blk((_RB, LD)), scalar, scalar],
        out_shape=[
            jax.ShapeDtypeStruct((N, LD), jnp.float32),
            jax.ShapeDtypeStruct((1, 1), jnp.float32),
            jax.ShapeDtypeStruct((1, 1), jnp.float32),
        ],
    )(p0, p1, x, eps, w1, w2, wmu, wlv, wd)


_CHC = 80
_ERC = E // _CHC
_BPCC = _ERC // NC // _BLK
_NLOC = _BPCC // NS
_MAXCHC = (_NLOC + 1) * _BLK


def _edge_dot_chunk(zs, zd, pbuf, lbuf, k):
    for e in range(_CHC):
        p0 = zs[e, pl.ds(0, 16)] * zd[e, pl.ds(0, 16)]
        p1 = zs[e, pl.ds(16, 16)] * zd[e, pl.ds(16, 16)]
        p2 = zs[e, pl.ds(32, 16)] * zd[e, pl.ds(32, 16)]
        p3 = zs[e, pl.ds(48, 16)] * zd[e, pl.ds(48, 16)]
        pbuf[e, pl.ds(0, 16)] = (p0 + p1) + (p2 + p3)
    for g in range(_CHC // LANES):
        rowi = g * LANES + lax.iota(jnp.int32, LANES)
        acc = plsc.load_gather(pbuf, [rowi, jnp.zeros((LANES,), jnp.int32)])
        for j in range(1, LANES):
            acc = acc + plsc.load_gather(pbuf, [rowi, jnp.full((LANES,), j, jnp.int32)])
        lbuf[pl.ds(k * _CHC + g * LANES, LANES)] = acc


def _edge_body(src2_hbm, dst2_hbm, z_hbm, logit_hbm,
               swin, dwin, zs0, zd0, zs1, zd1, pbuf, lbuf, g0, g1):
    c = lax.axis_index("c")
    s = lax.axis_index("s")
    b0, nb = _tile_range(c, s, _BPCC)
    nchunk = nb * _BLK
    _load_idx(src2_hbm, dst2_hbm, swin, dwin, b0, nb, _NLOC)

    pltpu.async_copy(z_hbm.at[swin.at[0]], zs0, g0)
    pltpu.async_copy(z_hbm.at[dwin.at[0]], zd0, g0)

    def chunk(k, carry):
        @pl.when(k % 2 == 0)
        def _():
            @pl.when(k + 1 < nchunk)
            def _():
                pltpu.async_copy(z_hbm.at[swin.at[k + 1]], zs1, g1)
                pltpu.async_copy(z_hbm.at[dwin.at[k + 1]], zd1, g1)

            pltpu.make_async_copy(z_hbm.at[swin.at[k]], zs0, g0).wait()
            pltpu.make_async_copy(z_hbm.at[dwin.at[k]], zd0, g0).wait()
            _edge_dot_chunk(zs0, zd0, pbuf, lbuf, k)

        @pl.when(k % 2 == 1)
        def _():
            @pl.when(k + 1 < nchunk)
            def _():
                pltpu.async_copy(z_hbm.at[swin.at[k + 1]], zs0, g0)
                pltpu.async_copy(z_hbm.at[dwin.at[k + 1]], zd0, g0)

            pltpu.make_async_copy(z_hbm.at[swin.at[k]], zs1, g1).wait()
            pltpu.make_async_copy(z_hbm.at[dwin.at[k]], zd1, g1).wait()
            _edge_dot_chunk(zs1, zd1, pbuf, lbuf, k)

        return carry

    lax.fori_loop(0, nchunk, chunk, 0)

    e0 = b0 * _BLK * _CHC
    nfix = _NLOC * _BLK * _CHC
    pltpu.sync_copy(lbuf.at[pl.ds(0, nfix)], logit_hbm.at[pl.ds(e0, nfix)])

    @pl.when(nchunk * _CHC > nfix)
    def _():
        pltpu.sync_copy(
            lbuf.at[pl.ds(nfix, _BLK * _CHC)],
            logit_hbm.at[pl.ds(e0 + nfix, _BLK * _CHC)],
        )


@functools.lru_cache(maxsize=None)
def _edge_call():
    return functools.partial(
        pl.kernel,
        out_type=jax.ShapeDtypeStruct((E,), jnp.float32),
        mesh=_sc_mesh(),
        compiler_params=pltpu.CompilerParams(
            needs_layout_passes=False, use_tc_tiling_on_sc=False
        ),
        scratch_types=[
            pltpu.VMEM((_MAXCHC, _CHC), jnp.int32),
            pltpu.VMEM((_MAXCHC, _CHC), jnp.int32),
            pltpu.VMEM((_CHC, LD), jnp.float32),
            pltpu.VMEM((_CHC, LD), jnp.float32),
            pltpu.VMEM((_CHC, LD), jnp.float32),
            pltpu.VMEM((_CHC, LD), jnp.float32),
            pltpu.VMEM((_CHC, 17), jnp.float32),
            pltpu.VMEM((_MAXCHC * _CHC,), jnp.float32),
            pltpu.SemaphoreType.DMA,
            pltpu.SemaphoreType.DMA,
        ],
    )(_edge_body)


def _tail_body(l_ref, kl_ref, nlp_ref, out_ref):
    t = l_ref[...]
    elp = jnp.sum(jnp.minimum(t, 0.0) - jnp.log1p(jnp.exp(-jnp.abs(t))))
    out_ref[0, 0] = -((nlp_ref[0, 0] + elp - kl_ref[0, 0]) / NUM_SEGMENTS)


def _tail_call(logits2d, kl_s, nlp_s):
    scalar = pl.BlockSpec(memory_space=pltpu.SMEM)
    return pl.pallas_call(
        _tail_body,
        in_specs=[pl.BlockSpec(logits2d.shape, lambda: (0, 0)), scalar, scalar],
        out_specs=scalar,
        out_shape=jax.ShapeDtypeStruct((1, 1), jnp.float32),
    )(logits2d, kl_s, nlp_s)


def kernel(x, edge_index, batch, eps, W1, W2, Wmu, Wlv, Wd):
    del batch
    src = edge_index[0]
    dst = edge_index[1]
    zeros = jnp.zeros((N, D), jnp.float32)
    parts = _agg_call()(src.reshape(_ERA, _CHA), dst.reshape(_ERA, _CHA), x, zeros)
    z, kl_s, nlp_s = _dense_call(parts[0], parts[1], x, eps, W1, W2, Wmu, Wlv, Wd)
    logits = _edge_call()(src.reshape(_ERC, _CHC), dst.reshape(_ERC, _CHC), z)
    out = _tail_call(logits.reshape(E // D, D), kl_s, nlp_s)
    return out[0, 0]


# speedup vs baseline: 2.4089x; 1.0657x over previous
import functools

import jax
import jax.numpy as jnp
from jax import lax
from jax.experimental import pallas as pl
from jax.experimental.pallas import tpu as pltpu
from jax.experimental.pallas import tpu_sc as plsc

N = 10000
E = 320000
D = 128
H = 256
LD = 64
NUM_SEGMENTS = 256.0

NC = 2
NS = 16
LANES = 16

LOG2PI = 1.8378770664093453


def _sc_mesh():
    return plsc.VectorSubcoreMesh(
        core_axis_name="c", subcore_axis_name="s", num_cores=NC, num_subcores=NS
    )


_EPC = E // NC
_BLK = 8
_RSTRIDE = 624
_RWIN = 640


def _tile_range(c, s, bpc):
    nhi = bpc % NS
    nlo = bpc // NS
    nb = jnp.where(s < nhi, nlo + 1, nlo)
    start = c * bpc + jnp.where(s < nhi, s * (nlo + 1), nhi * (nlo + 1) + (s - nhi) * nlo)
    return start, nb


def _load_idx(src2_hbm, dst2_hbm, swin, dwin, b0, nb, nlo):
    r0 = b0 * _BLK
    pltpu.sync_copy(src2_hbm.at[pl.ds(r0, nlo * _BLK)], swin.at[pl.ds(0, nlo * _BLK)])
    pltpu.sync_copy(dst2_hbm.at[pl.ds(r0, nlo * _BLK)], dwin.at[pl.ds(0, nlo * _BLK)])

    @pl.when(nb > nlo)
    def _():
        pltpu.sync_copy(src2_hbm.at[pl.ds(r0 + nlo * _BLK, _BLK)],
                        swin.at[pl.ds(nlo * _BLK, _BLK)])
        pltpu.sync_copy(dst2_hbm.at[pl.ds(r0 + nlo * _BLK, _BLK)],
                        dwin.at[pl.ds(nlo * _BLK, _BLK)])


_CHA = 40
_ERA = E // _CHA
_BPCA = _ERA // NC // _BLK
_NLOA = _BPCA // NS
_MAXCHA = (_NLOA + 1) * _BLK


def _agg_body(src2_hbm, dst2_hbm, x_hbm, zeros_hbm, out_hbm,
              swin, dwin, rows0, rows1, acc, g0, g1, s0, s1):
    c = lax.axis_index("c")
    s = lax.axis_index("s")
    pltpu.sync_copy(
        zeros_hbm.at[pl.ds(s * _RSTRIDE, _RWIN)], acc.at[pl.ds(s * _RSTRIDE, _RWIN)]
    )
    b0, nb = _tile_range(c, s, _BPCA)
    nchunk = nb * _BLK
    _load_idx(src2_hbm, dst2_hbm, swin, dwin, b0, nb, _NLOA)
    plsc.subcore_barrier()

    pltpu.async_copy(x_hbm.at[swin.at[0]], rows0, g0)

    def chunk(k, carry):
        @pl.when(k % 2 == 0)
        def _():
            @pl.when(k >= 1)
            def _():
                pltpu.make_async_copy(rows1, acc.at[dwin.at[k - 1]], s1).wait()

            @pl.when(k + 1 < nchunk)
            def _():
                pltpu.async_copy(x_hbm.at[swin.at[k + 1]], rows1, g1)

            pltpu.make_async_copy(x_hbm.at[swin.at[k]], rows0, g0).wait()
            pltpu.async_copy(rows0, acc.at[dwin.at[k]], s0, add=True)

        @pl.when(k % 2 == 1)
        def _():
            pltpu.make_async_copy(rows0, acc.at[dwin.at[k - 1]], s0).wait()

            @pl.when(k + 1 < nchunk)
            def _():
                pltpu.async_copy(x_hbm.at[swin.at[k + 1]], rows0, g0)

            pltpu.make_async_copy(x_hbm.at[swin.at[k]], rows1, g1).wait()
            pltpu.async_copy(rows1, acc.at[dwin.at[k]], s1, add=True)

        return carry

    lax.fori_loop(0, nchunk, chunk, 0)
    pltpu.make_async_copy(rows1, acc.at[dwin.at[nchunk - 1]], s1).wait()
    plsc.subcore_barrier()
    pltpu.sync_copy(
        acc.at[pl.ds(s * _RSTRIDE, _RWIN)], out_hbm.at[c, pl.ds(s * _RSTRIDE, _RWIN)]
    )


@functools.lru_cache(maxsize=None)
def _agg_call():
    return functools.partial(
        pl.kernel,
        out_type=jax.ShapeDtypeStruct((NC, N, D), jnp.float32),
        mesh=_sc_mesh(),
        compiler_params=pltpu.CompilerParams(
            needs_layout_passes=False, use_tc_tiling_on_sc=False
        ),
        scratch_types=[
            pltpu.VMEM((_MAXCHA, _CHA), jnp.int32),
            pltpu.VMEM((_MAXCHA, _CHA), jnp.int32),
            pltpu.VMEM((_CHA, D), jnp.float32),
            pltpu.VMEM((_CHA, D), jnp.float32),
            pltpu.VMEM_SHARED((N, D), jnp.float32),
            pltpu.SemaphoreType.DMA,
            pltpu.SemaphoreType.DMA,
            pltpu.SemaphoreType.DMA,
            pltpu.SemaphoreType.DMA,
        ],
    )(_agg_body)


_RB = 2000
_NB = N // _RB


def _dense_body(p0, p1, x, eps, w1, w2, wmu, wlv, wd, z_out, kl_out, nlp_out):
    i = pl.program_id(0)
    agg = p0[...] + p1[...]
    h = jnp.maximum(
        jnp.dot(agg, w1[...], preferred_element_type=jnp.float32)
        + jnp.dot(x[...], w2[...], preferred_element_type=jnp.float32),
        0.0,
    )
    mu = jnp.dot(h, wmu[...], preferred_element_type=jnp.float32)
    lv = jnp.clip(jnp.dot(h, wlv[...], preferred_element_type=jnp.float32), -8.0, 8.0)
    s2 = jnp.exp(lv)
    z = mu + jnp.exp(0.5 * lv) * eps[...]
    z_out[...] = jnp.concatenate([z, jnp.zeros_like(z)], axis=1)
    klb = 0.5 * jnp.sum(mu * mu + s2 - 1.0 - lv)
    xr = jnp.dot(z, wd[...], preferred_element_type=jnp.float32)
    nlb = -0.5 * jnp.sum((x[...] - xr) ** 2) - 0.5 * _RB * D * LOG2PI

    @pl.when(i == 0)
    def _():
        kl_out[0, 0] = klb
        nlp_out[0, 0] = nlb

    @pl.when(i != 0)
    def _():
        kl_out[0, 0] += klb
        nlp_out[0, 0] += nlb


def _dense_call(p0, p1, x, eps, w1, w2, wmu, wlv, wd):
    full = lambda shape: pl.BlockSpec(shape, lambda i: (0, 0))
    blk = lambda shape: pl.BlockSpec(shape, lambda i: (i, 0))
    scalar = pl.BlockSpec((1, 1), lambda i: (0, 0), memory_space=pltpu.SMEM)
    return pl.pallas_call(
        _dense_body,
        grid=(_NB,),
        in_specs=[
            blk((_RB, D)), blk((_RB, D)), blk((_RB, D)), blk((_RB, LD)),
            full((D, H)), full((D, H)), full((H, LD)), full((H, LD)), full((LD, D)),
        ],
        out_specs=[blk((_RB, 2 * LD)), scalar, scalar],
        out_shape=[
            jax.ShapeDtypeStruct((N, 2 * LD), jnp.float32),
            jax.ShapeDtypeStruct((1, 1), jnp.float32),
            jax.ShapeDtypeStruct((1, 1), jnp.float32),
        ],
    )(p0, p1, x, eps, w1, w2, wmu, wlv, wd)


_CHC = 80
_ERC = E // _CHC
_BPCC = _ERC // NC // _BLK
_NLOC = _BPCC // NS
_MAXCHC = (_NLOC + 1) * _BLK


def _edge_dot_chunk(zs, zd, pbuf, lbuf, k):
    for e in range(_CHC):
        p0 = zs[e, pl.ds(0, 16)] * zd[e, pl.ds(0, 16)]
        p1 = zs[e, pl.ds(16, 16)] * zd[e, pl.ds(16, 16)]
        p2 = zs[e, pl.ds(32, 16)] * zd[e, pl.ds(32, 16)]
        p3 = zs[e, pl.ds(48, 16)] * zd[e, pl.ds(48, 16)]
        pbuf[e, pl.ds(0, 16)] = (p0 + p1) + (p2 + p3)
    for g in range(_CHC // LANES):
        rowi = g * LANES + lax.iota(jnp.int32, LANES)
        acc = plsc.load_gather(pbuf, [rowi, jnp.zeros((LANES,), jnp.int32)])
        for j in range(1, LANES):
            acc = acc + plsc.load_gather(pbuf, [rowi, jnp.full((LANES,), j, jnp.int32)])
        lbuf[pl.ds(k * _CHC + g * LANES, LANES)] = acc


def _edge_body(src2_hbm, dst2_hbm, z_hbm, logit_hbm,
               swin, dwin, zs0, zd0, zs1, zd1, pbuf, lbuf, g0, g1):
    c = lax.axis_index("c")
    s = lax.axis_index("s")
    b0, nb = _tile_range(c, s, _BPCC)
    nchunk = nb * _BLK
    _load_idx(src2_hbm, dst2_hbm, swin, dwin, b0, nb, _NLOC)

    pltpu.async_copy(z_hbm.at[swin.at[0]], zs0, g0)
    pltpu.async_copy(z_hbm.at[dwin.at[0]], zd0, g0)

    def chunk(k, carry):
        @pl.when(k % 2 == 0)
        def _():
            @pl.when(k + 1 < nchunk)
            def _():
                pltpu.async_copy(z_hbm.at[swin.at[k + 1]], zs1, g1)
                pltpu.async_copy(z_hbm.at[dwin.at[k + 1]], zd1, g1)

            pltpu.make_async_copy(z_hbm.at[swin.at[k]], zs0, g0).wait()
            pltpu.make_async_copy(z_hbm.at[dwin.at[k]], zd0, g0).wait()
            _edge_dot_chunk(zs0, zd0, pbuf, lbuf, k)

        @pl.when(k % 2 == 1)
        def _():
            @pl.when(k + 1 < nchunk)
            def _():
                pltpu.async_copy(z_hbm.at[swin.at[k + 1]], zs0, g0)
                pltpu.async_copy(z_hbm.at[dwin.at[k + 1]], zd0, g0)

            pltpu.make_async_copy(z_hbm.at[swin.at[k]], zs1, g1).wait()
            pltpu.make_async_copy(z_hbm.at[dwin.at[k]], zd1, g1).wait()
            _edge_dot_chunk(zs1, zd1, pbuf, lbuf, k)

        return carry

    lax.fori_loop(0, nchunk, chunk, 0)

    e0 = b0 * _BLK * _CHC
    nfix = _NLOC * _BLK * _CHC
    pltpu.sync_copy(lbuf.at[pl.ds(0, nfix)], logit_hbm.at[pl.ds(e0, nfix)])

    @pl.when(nchunk * _CHC > nfix)
    def _():
        pltpu.sync_copy(
            lbuf.at[pl.ds(nfix, _BLK * _CHC)],
            logit_hbm.at[pl.ds(e0 + nfix, _BLK * _CHC)],
        )


@functools.lru_cache(maxsize=None)
def _edge_call():
    return functools.partial(
        pl.kernel,
        out_type=jax.ShapeDtypeStruct((E,), jnp.float32),
        mesh=_sc_mesh(),
        compiler_params=pltpu.CompilerParams(needs_layout_passes=False),
        scratch_types=[
            pltpu.VMEM((_MAXCHC, _CHC), jnp.int32),
            pltpu.VMEM((_MAXCHC, _CHC), jnp.int32),
            pltpu.VMEM((_CHC, 2 * LD), jnp.float32),
            pltpu.VMEM((_CHC, 2 * LD), jnp.float32),
            pltpu.VMEM((_CHC, 2 * LD), jnp.float32),
            pltpu.VMEM((_CHC, 2 * LD), jnp.float32),
            pltpu.VMEM((_CHC, 17), jnp.float32),
            pltpu.VMEM((_MAXCHC * _CHC,), jnp.float32),
            pltpu.SemaphoreType.DMA,
            pltpu.SemaphoreType.DMA,
        ],
    )(_edge_body)


def _tail_body(l_ref, kl_ref, nlp_ref, out_ref):
    t = l_ref[...]
    elp = jnp.sum(jnp.minimum(t, 0.0) - jnp.log1p(jnp.exp(-jnp.abs(t))))
    out_ref[0, 0] = -((nlp_ref[0, 0] + elp - kl_ref[0, 0]) / NUM_SEGMENTS)


def _tail_call(logits2d, kl_s, nlp_s):
    scalar = pl.BlockSpec(memory_space=pltpu.SMEM)
    return pl.pallas_call(
        _tail_body,
        in_specs=[pl.BlockSpec(logits2d.shape, lambda: (0, 0)), scalar, scalar],
        out_specs=scalar,
        out_shape=jax.ShapeDtypeStruct((1, 1), jnp.float32),
    )(logits2d, kl_s, nlp_s)


def kernel(x, edge_index, batch, eps, W1, W2, Wmu, Wlv, Wd):
    del batch
    src = edge_index[0]
    dst = edge_index[1]
    zeros = jnp.zeros((N, D), jnp.float32)
    parts = _agg_call()(src.reshape(_ERA, _CHA), dst.reshape(_ERA, _CHA), x, zeros)
    z, kl_s, nlp_s = _dense_call(parts[0], parts[1], x, eps, W1, W2, Wmu, Wlv, Wd)
    logits = _edge_call()(src.reshape(_ERC, _CHC), dst.reshape(_ERC, _CHC), z)
    out = _tail_call(logits.reshape(E // D, D), kl_s, nlp_s)
    return out[0, 0]
